# baseline (speedup 1.0000x reference)
import functools

import jax
import jax.numpy as jnp
from jax.experimental import pallas as pl
from jax.experimental.pallas import tpu as pltpu

_TOPK = 128
_NEG_CAP = -20.0


def _body(topk, e1_ref, e2_ref, a_ref, proxy_ref, logits_ref, cut_ref):
    e1 = e1_ref[...]
    e2 = e2_ref[...]
    f = jax.lax.dot_general(
        e1, e2, (((1,), (1,)), ((), ())), preferred_element_type=jnp.float32
    )
    b_rows, n = f.shape

    row_max_raw = jnp.max(f, axis=1, keepdims=True)
    row_max = jnp.maximum(row_max_raw, 0.0)
    lo0 = jnp.zeros((b_rows, 1), jnp.int32)
    hi0 = (jax.lax.bitcast_convert_type(row_max, jnp.int32)
           & jnp.int32(0x7FFFFFFF)) + 1
    cnt0 = jnp.full((b_rows, 1), n, jnp.int32)

    def _row_done(lo, hi, cnt_lo):
        return (cnt_lo == topk) | (hi - lo <= 1)

    halves = (f[: b_rows // 2], f[b_rows // 2:])

    def _probe(fh, lo, hi, cnt_lo, p):
        done = _row_done(lo, hi, cnt_lo)
        p = jnp.clip(p, lo + 1, hi - 1)
        pf = jax.lax.bitcast_convert_type(p, jnp.float32)
        cnt = jnp.sum((fh >= pf).astype(jnp.int32), axis=1, keepdims=True)
        ge = cnt >= topk
        lo2 = jnp.where(done | ~ge, lo, p)
        hi2 = jnp.where(done | ge, hi, p)
        cnt2 = jnp.where(done | ~ge, cnt_lo, cnt)
        return lo2, hi2, cnt2

    def _cond(carry):
        return jnp.logical_not(
            jnp.all(_row_done(carry[0], carry[1], carry[2]))
            & jnp.all(_row_done(carry[3], carry[4], carry[5])))

    def _mid(lo, hi):
        mid = 0.5 * (jax.lax.bitcast_convert_type(lo, jnp.float32)
                     + jax.lax.bitcast_convert_type(hi, jnp.float32))
        return jax.lax.bitcast_convert_type(mid, jnp.int32)

    def _step(carry):
        a = _probe(halves[0], carry[0], carry[1], carry[2],
                   _mid(carry[0], carry[1]))
        b = _probe(halves[1], carry[3], carry[4], carry[5],
                   _mid(carry[3], carry[4]))
        return a + b

    row_mean = jnp.sum(jnp.maximum(f, 0.0), axis=1, keepdims=True) * (1.0 / n)
    mh = (row_mean[: b_rows // 2], row_mean[b_rows // 2:])
    h = b_rows // 2
    carry = (lo0[:h], hi0[:h], cnt0[:h], lo0[h:], hi0[h:], cnt0[h:])
    for fac in (4.109, 5.229):
        a = _probe(halves[0], carry[0], carry[1], carry[2],
                   jax.lax.bitcast_convert_type(mh[0] * fac, jnp.int32))
        b = _probe(halves[1], carry[3], carry[4], carry[5],
                   jax.lax.bitcast_convert_type(mh[1] * fac, jnp.int32))
        carry = a + b

    res = jax.lax.while_loop(_cond, _step, carry)
    lo = jnp.concatenate([res[0], res[3]], axis=0)
    cnt_lo = jnp.concatenate([res[2], res[5]], axis=0)

    def _emit(sel, vals):
        ex = jnp.where(sel, jnp.exp(vals - row_max), 0.0)
        s = jnp.sum(ex, axis=1, keepdims=True)
        a_ref[...] = ex / s
        proxy_ref[...] = jax.nn.sigmoid(jnp.where(sel, vals, _NEG_CAP))
        logits_ref[...] = jnp.where(sel, vals, -jnp.inf)

    simple = jnp.all(cnt_lo == topk)

    @pl.when(simple)
    def _fast():
        lo_f = jax.lax.bitcast_convert_type(lo, jnp.float32)
        _emit(f >= lo_f, f)

    @pl.when(jnp.logical_not(simple))
    def _general():
        v = jnp.maximum(f, 0.0)
        bits = jax.lax.bitcast_convert_type(v, jnp.int32) & jnp.int32(0x7FFFFFFF)
        mmin = jnp.min(jnp.where(bits >= lo, bits, jnp.int32(0x7FFFFFFF)),
                       axis=1, keepdims=True)
        t = jnp.where(cnt_lo == topk, mmin, lo)

        gt = bits > t
        eq = bits == t
        c_gt = jnp.sum(gt.astype(jnp.int32), axis=1, keepdims=True)
        c_eq = jnp.sum(eq.astype(jnp.int32), axis=1, keepdims=True)
        need = topk - c_gt

        col = jax.lax.broadcasted_iota(jnp.int32, (b_rows, n), 1)
        cut_ref[...] = jnp.full((b_rows, 1), n - 1, jnp.int32)

        @pl.when(jnp.logical_not(jnp.all(c_eq == need)))
        def _tie_break():
            cut = jnp.zeros((b_rows, 1), jnp.int32)
            for b in range(11, -1, -1):
                cand = cut | jnp.int32(1 << b)
                cnt = jnp.sum((eq & (col < cand)).astype(jnp.int32), axis=1,
                              keepdims=True)
                cut = jnp.where(cnt < need, cand, cut)
            cut_ref[...] = cut

        _emit(gt | (eq & (col <= cut_ref[...])), v)


def kernel(E1, E2):
    n, emb = E1.shape
    block = 256
    grid = (n // block,)
    out = pl.pallas_call(
        functools.partial(_body, _TOPK),
        grid=grid,
        in_specs=[
            pl.BlockSpec((block, emb), lambda i: (i, 0)),
            pl.BlockSpec((n, emb), lambda i: (0, 0)),
        ],
        out_specs=[
            pl.BlockSpec((block, n), lambda i: (i, 0)),
            pl.BlockSpec((block, n), lambda i: (i, 0)),
            pl.BlockSpec((block, n), lambda i: (i, 0)),
        ],
        out_shape=[
            jax.ShapeDtypeStruct((n, n), jnp.float32),
            jax.ShapeDtypeStruct((n, n), jnp.float32),
            jax.ShapeDtypeStruct((n, n), jnp.float32),
        ],
        scratch_shapes=[pltpu.VMEM((block, 1), jnp.int32)],
    )(E1, E2)
    return tuple(out)


# speedup vs baseline: 1.0140x; 1.0140x over previous
import functools

import jax
import jax.numpy as jnp
from jax.experimental import pallas as pl
from jax.experimental.pallas import tpu as pltpu

_TOPK = 128
_NEG_CAP = -20.0


def _tree_reduce(parts, op):
    while len(parts) > 1:
        nxt = [op(parts[i], parts[i + 1]) for i in range(0, len(parts) - 1, 2)]
        if len(parts) % 2:
            nxt.append(parts[-1])
        parts = nxt
    if op is jnp.add:
        return jnp.sum(parts[0], axis=1, keepdims=True)
    if op is jnp.maximum:
        return jnp.max(parts[0], axis=1, keepdims=True)
    return jnp.min(parts[0], axis=1, keepdims=True)


def _body(topk, e1_ref, e2_ref, a_ref, proxy_ref, logits_ref, cut_ref):
    e1 = e1_ref[...]
    e2 = e2_ref[...]
    f = jax.lax.dot_general(
        e1, e2, (((1,), (1,)), ((), ())), preferred_element_type=jnp.float32
    )
    b_rows, n = f.shape

    row_max_raw = _tree_reduce(
        [f[:, j:j + 128] for j in range(0, n, 128)], jnp.maximum)
    row_max = jnp.maximum(row_max_raw, 0.0)
    lo0 = jnp.zeros((b_rows, 1), jnp.int32)
    hi0 = (jax.lax.bitcast_convert_type(row_max, jnp.int32)
           & jnp.int32(0x7FFFFFFF)) + 1
    cnt0 = jnp.full((b_rows, 1), n, jnp.int32)

    def _row_done(lo, hi, cnt_lo):
        return (cnt_lo == topk) | (hi - lo <= 1)

    def _probe(lo, hi, cnt_lo, p):
        done = _row_done(lo, hi, cnt_lo)
        p = jnp.clip(p, lo + 1, hi - 1)
        pf = jax.lax.bitcast_convert_type(p, jnp.float32)
        cnt = _tree_reduce(
            [(f[:, j:j + 128] >= pf).astype(jnp.int32)
             for j in range(0, n, 128)], jnp.add)
        ge = cnt >= topk
        lo2 = jnp.where(done | ~ge, lo, p)
        hi2 = jnp.where(done | ge, hi, p)
        cnt2 = jnp.where(done | ~ge, cnt_lo, cnt)
        return lo2, hi2, cnt2

    def _cond(carry):
        lo, hi, cnt_lo = carry
        return jnp.logical_not(jnp.all(_row_done(lo, hi, cnt_lo)))

    def _step(carry):
        lo, hi, cnt_lo = carry
        mid = 0.5 * (jax.lax.bitcast_convert_type(lo, jnp.float32)
                     + jax.lax.bitcast_convert_type(hi, jnp.float32))
        p = jax.lax.bitcast_convert_type(mid, jnp.int32)
        return _probe(lo, hi, cnt_lo, p)

    row_mean = _tree_reduce(
        [jnp.maximum(f[:, j:j + 128], 0.0) for j in range(0, n, 128)],
        jnp.add) * (1.0 / n)
    carry = (lo0, hi0, cnt0)
    for fac in (4.109, 5.229):
        p_guess = jax.lax.bitcast_convert_type(row_mean * fac, jnp.int32)
        carry = _probe(*carry, p_guess)

    lo, hi, cnt_lo = jax.lax.while_loop(_cond, _step, carry)

    def _emit(sel, vals):
        ex = jnp.where(sel, jnp.exp(vals - row_max), 0.0)
        s = _tree_reduce([ex[:, j:j + 128] for j in range(0, n, 128)],
                         jnp.add)
        a_ref[...] = ex / s
        proxy_ref[...] = jax.nn.sigmoid(jnp.where(sel, vals, _NEG_CAP))
        logits_ref[...] = jnp.where(sel, vals, -jnp.inf)

    simple = jnp.all(cnt_lo == topk)

    @pl.when(simple)
    def _fast():
        lo_f = jax.lax.bitcast_convert_type(lo, jnp.float32)
        _emit(f >= lo_f, f)

    @pl.when(jnp.logical_not(simple))
    def _general():
        v = jnp.maximum(f, 0.0)
        bits = jax.lax.bitcast_convert_type(v, jnp.int32) & jnp.int32(0x7FFFFFFF)
        mmin = jnp.min(jnp.where(bits >= lo, bits, jnp.int32(0x7FFFFFFF)),
                       axis=1, keepdims=True)
        t = jnp.where(cnt_lo == topk, mmin, lo)

        gt = bits > t
        eq = bits == t
        c_gt = jnp.sum(gt.astype(jnp.int32), axis=1, keepdims=True)
        c_eq = jnp.sum(eq.astype(jnp.int32), axis=1, keepdims=True)
        need = topk - c_gt

        col = jax.lax.broadcasted_iota(jnp.int32, (b_rows, n), 1)
        cut_ref[...] = jnp.full((b_rows, 1), n - 1, jnp.int32)

        @pl.when(jnp.logical_not(jnp.all(c_eq == need)))
        def _tie_break():
            cut = jnp.zeros((b_rows, 1), jnp.int32)
            for b in range(11, -1, -1):
                cand = cut | jnp.int32(1 << b)
                cnt = jnp.sum((eq & (col < cand)).astype(jnp.int32), axis=1,
                              keepdims=True)
                cut = jnp.where(cnt < need, cand, cut)
            cut_ref[...] = cut

        _emit(gt | (eq & (col <= cut_ref[...])), v)


def kernel(E1, E2):
    n, emb = E1.shape
    block = 256
    grid = (n // block,)
    out = pl.pallas_call(
        functools.partial(_body, _TOPK),
        grid=grid,
        in_specs=[
            pl.BlockSpec((block, emb), lambda i: (i, 0)),
            pl.BlockSpec((n, emb), lambda i: (0, 0)),
        ],
        out_specs=[
            pl.BlockSpec((block, n), lambda i: (i, 0)),
            pl.BlockSpec((block, n), lambda i: (i, 0)),
            pl.BlockSpec((block, n), lambda i: (i, 0)),
        ],
        out_shape=[
            jax.ShapeDtypeStruct((n, n), jnp.float32),
            jax.ShapeDtypeStruct((n, n), jnp.float32),
            jax.ShapeDtypeStruct((n, n), jnp.float32),
        ],
        scratch_shapes=[pltpu.VMEM((block, 1), jnp.int32)],
    )(E1, E2)
    return tuple(out)
